# baseline (device time: 25396 ns/iter reference)
import functools

import jax
import jax.numpy as jnp
from jax import lax
from jax.experimental import pallas as pl
from jax.experimental.pallas import tpu as pltpu

N_DEV = 4
B_LOC = 2
SQ = 256
SKV = 256
HQ = 16
H_CHUNK = 4
DH = 64
D_MODEL = 512
WIN = 128

_sem_signal = getattr(pl, "semaphore_signal", None) or pltpu.semaphore_signal
_sem_wait = getattr(pl, "semaphore_wait", None) or pltpu.semaphore_wait
_DevIdType = getattr(pl, "DeviceIdType", None) or pltpu.DeviceIdType
_CompilerParams = getattr(pltpu, "CompilerParams", None) or pltpu.TPUCompilerParams


def _body(x_ref, wq_ref, kt_ref, v_ref, wo_ref, out_ref,
          wq_all, wo_all, ctx_sc, x16,
          wq_ssem, wo_ssem, wq_rsem, wo_rsem):
    my = lax.axis_index("i")

    wq_all[my] = wq_ref[...].astype(jnp.bfloat16)
    wo_all[my] = wo_ref[...].astype(jnp.bfloat16)
    x16[...] = x_ref[...].astype(jnp.bfloat16)

    barrier = pltpu.get_barrier_semaphore()
    for d in range(1, N_DEV):
        _sem_signal(barrier, inc=1, device_id=((my + d) % N_DEV,),
                    device_id_type=_DevIdType.MESH)
    _sem_wait(barrier, N_DEV - 1)

    send_ops = []
    for d in (3, 1, 2):
        tgt = (my + d) % N_DEV
        for buf, ssem, rsem in ((wq_all, wq_ssem, wq_rsem),
                                (wo_all, wo_ssem, wo_rsem)):
            op = pltpu.make_async_remote_copy(
                src_ref=buf.at[my], dst_ref=buf.at[my],
                send_sem=ssem.at[d - 1], recv_sem=rsem.at[my],
                device_id=(tgt,), device_id_type=_DevIdType.MESH)
            op.start()
            send_ops.append(op)

    qi = lax.broadcasted_iota(jnp.int32, (SQ, SKV), 0)
    ki = lax.broadcasted_iota(jnp.int32, (SQ, SKV), 1)
    mask = jnp.abs(qi - ki) <= WIN

    def attn_chunk(c):
        wq_c = wq_all[c]
        for b in range(B_LOC):
            q = lax.dot_general(
                x16[b], wq_c, (((1,), (0,)), ((), ())),
                preferred_element_type=jnp.float32,
            ).astype(jnp.bfloat16)
            for j in range(H_CHUNK):
                kt = kt_ref[b * HQ + c * H_CHUNK + j]
                vh = v_ref[b * HQ + c * H_CHUNK + j]
                qh = q[:, j * DH:(j + 1) * DH]
                s = lax.dot_general(
                    qh, kt, (((1,), (0,)), ((), ())),
                    preferred_element_type=jnp.float32) * 0.125
                w = jnp.where(mask, jnp.exp(s), 0.0)
                rsum = 1.0 / jnp.sum(w, axis=1, keepdims=True)
                ctx = lax.dot_general(
                    w.astype(jnp.bfloat16), vh, (((1,), (0,)), ((), ())),
                    preferred_element_type=jnp.float32)
                ctx_sc[b, :, j * DH:(j + 1) * DH] = (
                    ctx * rsum).astype(jnp.bfloat16)

    def project_chunk(c, first=False):
        wo_c = wo_all[c]
        for b in range(B_LOC):
            part = lax.dot_general(
                ctx_sc[b], wo_c, (((1,), (0,)), ((), ())),
                preferred_element_type=jnp.float32)
            out_ref[b] = part if first else out_ref[b] + part

    def wait_chunk(src, buf, ssem, rsem):
        pltpu.make_async_remote_copy(
            src_ref=buf.at[src], dst_ref=buf.at[src],
            send_sem=ssem.at[0], recv_sem=rsem.at[src],
            device_id=(src,), device_id_type=_DevIdType.MESH).wait_recv()

    attn_chunk(my)
    project_chunk(my, first=True)
    for d in (1, 3, 2):
        src = (my + d) % N_DEV
        wait_chunk(src, wq_all, wq_ssem, wq_rsem)
        attn_chunk(src)
        wait_chunk(src, wo_all, wo_ssem, wo_rsem)
        project_chunk(src)

    for op in send_ops:
        op.wait_send()


def kernel(x, Wq, K_ext, V_ext, Wo):
    my = lax.axis_index("i")
    k_sl = lax.dynamic_slice_in_dim(K_ext, my * B_LOC, B_LOC, 0)
    v_sl = lax.dynamic_slice_in_dim(V_ext, my * B_LOC, B_LOC, 0)
    k_t = jnp.transpose(k_sl, (0, 2, 3, 1)).reshape(B_LOC * HQ, DH, SKV)
    v_t = jnp.transpose(v_sl, (0, 2, 1, 3)).reshape(B_LOC * HQ, SKV, DH)

    kt16 = k_t.astype(jnp.bfloat16)
    v16 = v_t.astype(jnp.bfloat16)

    return pl.pallas_call(
        _body,
        out_shape=jax.ShapeDtypeStruct((B_LOC, SQ, D_MODEL), jnp.float32),
        in_specs=[pl.BlockSpec(memory_space=pltpu.VMEM)] * 5,
        out_specs=pl.BlockSpec(memory_space=pltpu.VMEM),
        scratch_shapes=[
            pltpu.VMEM((N_DEV, D_MODEL, H_CHUNK * DH), jnp.bfloat16),
            pltpu.VMEM((N_DEV, H_CHUNK * DH, D_MODEL), jnp.bfloat16),
            pltpu.VMEM((B_LOC, SQ, H_CHUNK * DH), jnp.bfloat16),
            pltpu.VMEM((B_LOC, SQ, D_MODEL), jnp.bfloat16),
            pltpu.SemaphoreType.DMA((N_DEV - 1,)),
            pltpu.SemaphoreType.DMA((N_DEV - 1,)),
            pltpu.SemaphoreType.DMA((N_DEV,)),
            pltpu.SemaphoreType.DMA((N_DEV,)),
        ],
        compiler_params=_CompilerParams(collective_id=0),
    )(x, Wq, kt16, v16, Wo)


# device time: 24838 ns/iter; 1.0225x vs baseline; 1.0225x over previous
import functools

import jax
import jax.numpy as jnp
from jax import lax
from jax.experimental import pallas as pl
from jax.experimental.pallas import tpu as pltpu

N_DEV = 4
B_LOC = 2
SQ = 256
SKV = 256
HQ = 16
H_CHUNK = 4
DH = 64
D_MODEL = 512
WIN = 128

_sem_signal = getattr(pl, "semaphore_signal", None) or pltpu.semaphore_signal
_sem_wait = getattr(pl, "semaphore_wait", None) or pltpu.semaphore_wait
_DevIdType = getattr(pl, "DeviceIdType", None) or pltpu.DeviceIdType
_CompilerParams = getattr(pltpu, "CompilerParams", None) or pltpu.TPUCompilerParams


def _body(x_ref, wq_ref, kt_ref, v_ref, wo_ref, out_ref,
          wq_all, wo_all, ctx_sc, x16,
          wq_ssem, wo_ssem, wq_rsem, wo_rsem):
    my = lax.axis_index("i")

    barrier = pltpu.get_barrier_semaphore()
    for d in range(1, N_DEV):
        _sem_signal(barrier, inc=1, device_id=((my + d) % N_DEV,),
                    device_id_type=_DevIdType.MESH)
    _sem_wait(barrier, N_DEV - 1)

    wq_all[my] = wq_ref[...].astype(jnp.bfloat16)
    wo_all[my] = wo_ref[...].astype(jnp.bfloat16)
    send_ops = []
    for d in (3, 1, 2):
        tgt = (my + d) % N_DEV
        for buf, ssem, rsem in ((wq_all, wq_ssem, wq_rsem),
                                (wo_all, wo_ssem, wo_rsem)):
            op = pltpu.make_async_remote_copy(
                src_ref=buf.at[my], dst_ref=buf.at[my],
                send_sem=ssem.at[d - 1], recv_sem=rsem.at[my],
                device_id=(tgt,), device_id_type=_DevIdType.MESH)
            op.start()
            send_ops.append(op)

    qi = lax.broadcasted_iota(jnp.int32, (SQ, SKV), 0)
    ki = lax.broadcasted_iota(jnp.int32, (SQ, SKV), 1)
    mask = jnp.abs(qi - ki) <= WIN

    x16[...] = x_ref[...].astype(jnp.bfloat16)

    def attn_chunk(c):
        wq_c = wq_all[c]
        for b in range(B_LOC):
            q = lax.dot_general(
                x16[b], wq_c, (((1,), (0,)), ((), ())),
                preferred_element_type=jnp.float32,
            ).astype(jnp.bfloat16)
            for j in range(H_CHUNK):
                kt = kt_ref[b * HQ + c * H_CHUNK + j]
                vh = v_ref[b * HQ + c * H_CHUNK + j]
                qh = q[:, j * DH:(j + 1) * DH]
                s = lax.dot_general(
                    qh, kt, (((1,), (0,)), ((), ())),
                    preferred_element_type=jnp.float32) * 0.125
                w = jnp.where(mask, jnp.exp(s), 0.0)
                rsum = 1.0 / jnp.sum(w, axis=1, keepdims=True)
                ctx = lax.dot_general(
                    w.astype(jnp.bfloat16), vh, (((1,), (0,)), ((), ())),
                    preferred_element_type=jnp.float32)
                ctx_sc[b, :, j * DH:(j + 1) * DH] = (
                    ctx * rsum).astype(jnp.bfloat16)

    def project_chunk(c, first=False):
        wo_c = wo_all[c]
        for b in range(B_LOC):
            part = lax.dot_general(
                ctx_sc[b], wo_c, (((1,), (0,)), ((), ())),
                preferred_element_type=jnp.float32)
            out_ref[b] = part if first else out_ref[b] + part

    def wait_chunk(src, buf, ssem, rsem):
        pltpu.make_async_remote_copy(
            src_ref=buf.at[src], dst_ref=buf.at[src],
            send_sem=ssem.at[0], recv_sem=rsem.at[src],
            device_id=(src,), device_id_type=_DevIdType.MESH).wait_recv()

    attn_chunk(my)
    project_chunk(my, first=True)
    for d in (1, 3, 2):
        src = (my + d) % N_DEV
        wait_chunk(src, wq_all, wq_ssem, wq_rsem)
        attn_chunk(src)
        wait_chunk(src, wo_all, wo_ssem, wo_rsem)
        project_chunk(src)

    for op in send_ops:
        op.wait_send()


def kernel(x, Wq, K_ext, V_ext, Wo):
    my = lax.axis_index("i")
    k_sl = lax.dynamic_slice_in_dim(K_ext, my * B_LOC, B_LOC, 0)
    v_sl = lax.dynamic_slice_in_dim(V_ext, my * B_LOC, B_LOC, 0)
    k_t = jnp.transpose(k_sl, (0, 2, 3, 1)).reshape(B_LOC * HQ, DH, SKV)
    v_t = jnp.transpose(v_sl, (0, 2, 1, 3)).reshape(B_LOC * HQ, SKV, DH)

    kt16 = k_t.astype(jnp.bfloat16)
    v16 = v_t.astype(jnp.bfloat16)

    return pl.pallas_call(
        _body,
        out_shape=jax.ShapeDtypeStruct((B_LOC, SQ, D_MODEL), jnp.float32),
        in_specs=[pl.BlockSpec(memory_space=pltpu.VMEM)] * 5,
        out_specs=pl.BlockSpec(memory_space=pltpu.VMEM),
        scratch_shapes=[
            pltpu.VMEM((N_DEV, D_MODEL, H_CHUNK * DH), jnp.bfloat16),
            pltpu.VMEM((N_DEV, H_CHUNK * DH, D_MODEL), jnp.bfloat16),
            pltpu.VMEM((B_LOC, SQ, H_CHUNK * DH), jnp.bfloat16),
            pltpu.VMEM((B_LOC, SQ, D_MODEL), jnp.bfloat16),
            pltpu.SemaphoreType.DMA((N_DEV - 1,)),
            pltpu.SemaphoreType.DMA((N_DEV - 1,)),
            pltpu.SemaphoreType.DMA((N_DEV,)),
            pltpu.SemaphoreType.DMA((N_DEV,)),
        ],
        compiler_params=_CompilerParams(collective_id=0),
    )(x, Wq, kt16, v16, Wo)
